# baseline (speedup 1.0000x reference)
import jax
import jax.numpy as jnp
from jax.experimental import pallas as pl
from jax.experimental.pallas import tpu as pltpu

_IMG = 512.0
_CONF_T = 0.5
_IOU_T = 0.35
_NUM_PRED = 10
_NCLS = 81
_N = 20000
_CHUNK = 8192
_NCHUNK = 3
_ROWS = 192
_NEG_INF = float("-inf")


def _body(y_ref, o_ref, sc_s, x1_s, y1_s, x2_s, y2_s, t_s):
    j = pl.program_id(1)
    liota = jax.lax.broadcasted_iota(jnp.int32, (1, 128), 1)

    _LAST_TILES = 157 - (_NCHUNK - 1) * (_CHUNK // 128)

    def _tile(k):
        yk = y_ref[0, k * 128:(k + 1) * 128, :]
        t = jnp.swapaxes(yk, 0, 1)
        row = j * (_CHUNK // 128) + k
        t_s[pl.ds(row, 1), :, 0:93] = yk.reshape(1, 128, 93)

        s0 = t[0:1, :]
        m_rest = jnp.max(t[1:_NCLS, :], axis=0, keepdims=True)
        conf = jnp.maximum(m_rest, s0)

        ocx = t[81:82, :]
        ocy = t[82:83, :]
        ow = t[83:84, :]
        oh = t[84:85, :]
        acx = t[85:86, :]
        acy = t[86:87, :]
        aw = t[87:88, :]
        ah = t[88:89, :]
        v0 = t[89:90, :]
        v1 = t[90:91, :]
        v2 = t[91:92, :]
        v3 = t[92:93, :]

        cx = ocx * v0 * aw + acx
        cy = ocy * v1 * ah + acy
        w = jnp.exp(ow * v2) * aw
        h = jnp.exp(oh * v3) * ah

        fl = j * _CHUNK + k * 128 + liota
        valid = (m_rest > s0) & (conf >= _CONF_T) & (fl < _N)
        scores = jnp.where(valid, conf, _NEG_INF)

        sc_s[pl.ds(row, 1), :] = scores
        x1_s[pl.ds(row, 1), :] = (cx - 0.5 * w) * _IMG
        y1_s[pl.ds(row, 1), :] = (cy - 0.5 * h) * _IMG
        x2_s[pl.ds(row, 1), :] = (cx + 0.5 * w) * _IMG
        y2_s[pl.ds(row, 1), :] = (cy + 0.5 * h) * _IMG

    for k in range(_CHUNK // 128):
        if k < _LAST_TILES:
            _tile(k)
        else:
            pl.when(j < _NCHUNK - 1)(lambda k=k: _tile(k))

    @pl.when(j == _NCHUNK - 1)
    def _():
        shape = (_ROWS, 128)
        flat0 = (jax.lax.broadcasted_iota(jnp.int32, shape, 0) * 128
                 + jax.lax.broadcasted_iota(jnp.int32, shape, 1))
        scores = jnp.where(flat0 < _N, sc_s[:, :], _NEG_INF)
        xmin = x1_s[:, :]
        ymin = y1_s[:, :]
        xmax = x2_s[:, :]
        ymax = y2_s[:, :]
        area = (jnp.maximum(xmax - xmin, 0.0)
                * jnp.maximum(ymax - ymin, 0.0))

        flat = (jax.lax.broadcasted_iota(jnp.int32, shape, 0) * 128
                + jax.lax.broadcasted_iota(jnp.int32, shape, 1))
        sub16 = jax.lax.broadcasted_iota(jnp.int32, (16, 128), 0)
        lane16 = jax.lax.broadcasted_iota(jnp.int32, (16, 128), 1)
        lane93 = jax.lax.broadcasted_iota(jnp.int32, (1, 93), 1)
        out_acc = jnp.zeros((16, 128), jnp.float32)

        for t in range(_NUM_PRED):
            m = jnp.max(scores)
            ok = m > _NEG_INF
            okf = jnp.where(ok, 1.0, 0.0).astype(jnp.float32)
            i = jnp.min(jnp.where(scores == m, flat, jnp.int32(2 ** 30)))
            sel = flat == i

            frow = t_s[i // 128, pl.ds(i % 128, 1), 0:93]
            bcls = jnp.min(jnp.where((frow == m) & (lane93 < _NCLS),
                                     lane93, 127)).astype(jnp.float32)
            ocx = frow[0:1, 81:82]
            ocy = frow[0:1, 82:83]
            ow = frow[0:1, 83:84]
            oh = frow[0:1, 84:85]
            acx = frow[0:1, 85:86]
            acy = frow[0:1, 86:87]
            aw = frow[0:1, 87:88]
            ah = frow[0:1, 88:89]
            v0 = frow[0:1, 89:90]
            v1 = frow[0:1, 90:91]
            v2 = frow[0:1, 91:92]
            v3 = frow[0:1, 92:93]
            cx = ocx * v0 * aw + acx
            cy = ocy * v1 * ah + acy
            w = jnp.exp(ow * v2) * aw
            h = jnp.exp(oh * v3) * ah
            bx1 = (cx - 0.5 * w) * _IMG
            by1 = (cy - 0.5 * h) * _IMG
            bx2 = (cx + 0.5 * w) * _IMG
            by2 = (cy + 0.5 * h) * _IMG

            row = (jnp.where(lane16 == 0, bcls, 0.0)
                   + jnp.where(lane16 == 1, m, 0.0)
                   + jnp.where(lane16 == 2, bx1, 0.0)
                   + jnp.where(lane16 == 3, by1, 0.0)
                   + jnp.where(lane16 == 4, bx2, 0.0)
                   + jnp.where(lane16 == 5, by2, 0.0))
            out_acc = out_acc + okf * jnp.where(sub16 == t, row, 0.0)

            ix1 = jnp.maximum(xmin, bx1)
            iy1 = jnp.maximum(ymin, by1)
            ix2 = jnp.minimum(xmax, bx2)
            iy2 = jnp.minimum(ymax, by2)
            inter = (jnp.maximum(ix2 - ix1, 0.0)
                     * jnp.maximum(iy2 - iy1, 0.0))
            barea = (jnp.maximum(bx2 - bx1, 0.0)
                     * jnp.maximum(by2 - by1, 0.0))
            iou = inter / jnp.maximum(area + barea - inter, 1e-8)
            supp = ((iou > _IOU_T) | sel) & ok
            scores = jnp.where(supp, _NEG_INF, scores)

        o_ref[0] = out_acc


def kernel(y_pred):
    b, n, d = y_pred.shape
    out = pl.pallas_call(
        _body,
        grid=(b, _NCHUNK),
        in_specs=[pl.BlockSpec((1, _CHUNK, d), lambda i, j: (i, j, 0))],
        out_specs=pl.BlockSpec((1, 16, 128), lambda i, j: (i, 0, 0)),
        out_shape=jax.ShapeDtypeStruct((b, 16, 128), jnp.float32),
        scratch_shapes=[
            pltpu.VMEM((_ROWS, 128), jnp.float32),
            pltpu.VMEM((_ROWS, 128), jnp.float32),
            pltpu.VMEM((_ROWS, 128), jnp.float32),
            pltpu.VMEM((_ROWS, 128), jnp.float32),
            pltpu.VMEM((_ROWS, 128), jnp.float32),
            pltpu.VMEM((_ROWS, 128, 96), jnp.float32),
        ],
        compiler_params=pltpu.CompilerParams(
            dimension_semantics=("arbitrary", "arbitrary")),
    )(y_pred)
    return out[:, :_NUM_PRED, :6]


# speedup vs baseline: 1.0670x; 1.0670x over previous
import jax
import jax.numpy as jnp
from jax.experimental import pallas as pl
from jax.experimental.pallas import tpu as pltpu

_IMG = 512.0
_CONF_T = 0.5
_IOU_T = 0.35
_NUM_PRED = 10
_NCLS = 81
_N = 20000
_CHUNK = 8192
_NCHUNK = 3
_ROWS = 192
_NEG_INF = float("-inf")


def _body(y_ref, o_ref, sc_s, x1_s, y1_s, x2_s, y2_s, t_s):
    j = pl.program_id(1)
    liota = jax.lax.broadcasted_iota(jnp.int32, (1, 128), 1)

    _LAST_TILES = 157 - (_NCHUNK - 1) * (_CHUNK // 128)

    def _tile(k):
        yk = y_ref[0, k * 128:(k + 1) * 128, :]
        t = jnp.swapaxes(yk, 0, 1)
        row = j * (_CHUNK // 128) + k
        t_s[pl.ds(row, 1), 0:93, :] = t.reshape(1, 93, 128)

        s0 = t[0:1, :]
        m_rest = jnp.max(t[1:_NCLS, :], axis=0, keepdims=True)
        conf = jnp.maximum(m_rest, s0)

        ocx = t[81:82, :]
        ocy = t[82:83, :]
        ow = t[83:84, :]
        oh = t[84:85, :]
        acx = t[85:86, :]
        acy = t[86:87, :]
        aw = t[87:88, :]
        ah = t[88:89, :]
        v0 = t[89:90, :]
        v1 = t[90:91, :]
        v2 = t[91:92, :]
        v3 = t[92:93, :]

        cx = ocx * v0 * aw + acx
        cy = ocy * v1 * ah + acy
        w = jnp.exp(ow * v2) * aw
        h = jnp.exp(oh * v3) * ah

        fl = j * _CHUNK + k * 128 + liota
        valid = (m_rest > s0) & (conf >= _CONF_T) & (fl < _N)
        scores = jnp.where(valid, conf, _NEG_INF)

        sc_s[pl.ds(row, 1), :] = scores
        x1_s[pl.ds(row, 1), :] = (cx - 0.5 * w) * _IMG
        y1_s[pl.ds(row, 1), :] = (cy - 0.5 * h) * _IMG
        x2_s[pl.ds(row, 1), :] = (cx + 0.5 * w) * _IMG
        y2_s[pl.ds(row, 1), :] = (cy + 0.5 * h) * _IMG

    for k in range(_CHUNK // 128):
        if k < _LAST_TILES:
            _tile(k)
        else:
            pl.when(j < _NCHUNK - 1)(lambda k=k: _tile(k))

    @pl.when(j == _NCHUNK - 1)
    def _():
        shape = (_ROWS, 128)
        flat0 = (jax.lax.broadcasted_iota(jnp.int32, shape, 0) * 128
                 + jax.lax.broadcasted_iota(jnp.int32, shape, 1))
        scores = jnp.where(flat0 < _N, sc_s[:, :], _NEG_INF)
        xmin = x1_s[:, :]
        ymin = y1_s[:, :]
        xmax = x2_s[:, :]
        ymax = y2_s[:, :]
        area = (jnp.maximum(xmax - xmin, 0.0)
                * jnp.maximum(ymax - ymin, 0.0))

        flat = (jax.lax.broadcasted_iota(jnp.int32, shape, 0) * 128
                + jax.lax.broadcasted_iota(jnp.int32, shape, 1))
        sub16 = jax.lax.broadcasted_iota(jnp.int32, (16, 128), 0)
        lane16 = jax.lax.broadcasted_iota(jnp.int32, (16, 128), 1)
        sub96 = jax.lax.broadcasted_iota(jnp.int32, (96, 128), 0)
        lane96 = jax.lax.broadcasted_iota(jnp.int32, (96, 128), 1)
        out_acc = jnp.zeros((16, 128), jnp.float32)

        for t in range(_NUM_PRED):
            m = jnp.max(scores)
            ok = m > _NEG_INF
            okf = jnp.where(ok, 1.0, 0.0).astype(jnp.float32)
            i = jnp.min(jnp.where(scores == m, flat, jnp.int32(2 ** 30)))
            sel = flat == i

            def ext(x):
                return jnp.sum(jnp.where(sel, x, 0.0))

            bx1 = ext(xmin)
            by1 = ext(ymin)
            bx2 = ext(xmax)
            by2 = ext(ymax)

            tile = t_s[i // 128]
            eqc = (tile == m) & (lane96 == i % 128) & (sub96 < _NCLS)
            bcls = jnp.min(jnp.where(eqc, sub96, 127)).astype(jnp.float32)

            row = (jnp.where(lane16 == 0, bcls, 0.0)
                   + jnp.where(lane16 == 1, m, 0.0)
                   + jnp.where(lane16 == 2, bx1, 0.0)
                   + jnp.where(lane16 == 3, by1, 0.0)
                   + jnp.where(lane16 == 4, bx2, 0.0)
                   + jnp.where(lane16 == 5, by2, 0.0))
            out_acc = out_acc + okf * jnp.where(sub16 == t, row, 0.0)

            ix1 = jnp.maximum(xmin, bx1)
            iy1 = jnp.maximum(ymin, by1)
            ix2 = jnp.minimum(xmax, bx2)
            iy2 = jnp.minimum(ymax, by2)
            inter = (jnp.maximum(ix2 - ix1, 0.0)
                     * jnp.maximum(iy2 - iy1, 0.0))
            barea = (jnp.maximum(bx2 - bx1, 0.0)
                     * jnp.maximum(by2 - by1, 0.0))
            iou = inter / jnp.maximum(area + barea - inter, 1e-8)
            supp = ((iou > _IOU_T) | sel) & ok
            scores = jnp.where(supp, _NEG_INF, scores)

        o_ref[0] = out_acc


def kernel(y_pred):
    b, n, d = y_pred.shape
    out = pl.pallas_call(
        _body,
        grid=(b, _NCHUNK),
        in_specs=[pl.BlockSpec((1, _CHUNK, d), lambda i, j: (i, j, 0))],
        out_specs=pl.BlockSpec((1, 16, 128), lambda i, j: (i, 0, 0)),
        out_shape=jax.ShapeDtypeStruct((b, 16, 128), jnp.float32),
        scratch_shapes=[
            pltpu.VMEM((_ROWS, 128), jnp.float32),
            pltpu.VMEM((_ROWS, 128), jnp.float32),
            pltpu.VMEM((_ROWS, 128), jnp.float32),
            pltpu.VMEM((_ROWS, 128), jnp.float32),
            pltpu.VMEM((_ROWS, 128), jnp.float32),
            pltpu.VMEM((_ROWS, 96, 128), jnp.float32),
        ],
        compiler_params=pltpu.CompilerParams(
            dimension_semantics=("arbitrary", "arbitrary")),
    )(y_pred)
    return out[:, :_NUM_PRED, :6]


# speedup vs baseline: 1.0877x; 1.0194x over previous
import jax
import jax.numpy as jnp
from jax.experimental import pallas as pl
from jax.experimental.pallas import tpu as pltpu

_IMG = 512.0
_CONF_T = 0.5
_IOU_T = 0.35
_NUM_PRED = 10
_NCLS = 81
_N = 20000
_CHUNK = 10240
_NCHUNK = 2
_ROWS = 160
_NEG_INF = float("-inf")


def _body(y_ref, o_ref, sc_s, x1_s, y1_s, x2_s, y2_s, t_s):
    j = pl.program_id(1)
    liota = jax.lax.broadcasted_iota(jnp.int32, (1, 128), 1)

    _LAST_TILES = 157 - (_NCHUNK - 1) * (_CHUNK // 128)

    def _tile(k):
        yk = y_ref[0, k * 128:(k + 1) * 128, :]
        t = jnp.swapaxes(yk, 0, 1)
        row = j * (_CHUNK // 128) + k
        t_s[pl.ds(row, 1), 0:93, :] = t.reshape(1, 93, 128)

        s0 = t[0:1, :]
        m_rest = jnp.max(t[1:_NCLS, :], axis=0, keepdims=True)
        conf = jnp.maximum(m_rest, s0)

        ocx = t[81:82, :]
        ocy = t[82:83, :]
        ow = t[83:84, :]
        oh = t[84:85, :]
        acx = t[85:86, :]
        acy = t[86:87, :]
        aw = t[87:88, :]
        ah = t[88:89, :]
        v0 = t[89:90, :]
        v1 = t[90:91, :]
        v2 = t[91:92, :]
        v3 = t[92:93, :]

        cx = ocx * v0 * aw + acx
        cy = ocy * v1 * ah + acy
        w = jnp.exp(ow * v2) * aw
        h = jnp.exp(oh * v3) * ah

        fl = j * _CHUNK + k * 128 + liota
        valid = (m_rest > s0) & (conf >= _CONF_T) & (fl < _N)
        scores = jnp.where(valid, conf, _NEG_INF)

        sc_s[pl.ds(row, 1), :] = scores
        x1_s[pl.ds(row, 1), :] = (cx - 0.5 * w) * _IMG
        y1_s[pl.ds(row, 1), :] = (cy - 0.5 * h) * _IMG
        x2_s[pl.ds(row, 1), :] = (cx + 0.5 * w) * _IMG
        y2_s[pl.ds(row, 1), :] = (cy + 0.5 * h) * _IMG

    for k in range(_CHUNK // 128):
        if k < _LAST_TILES:
            _tile(k)
        else:
            pl.when(j < _NCHUNK - 1)(lambda k=k: _tile(k))

    @pl.when(j == _NCHUNK - 1)
    def _():
        shape = (_ROWS, 128)
        flat0 = (jax.lax.broadcasted_iota(jnp.int32, shape, 0) * 128
                 + jax.lax.broadcasted_iota(jnp.int32, shape, 1))
        scores = jnp.where(flat0 < _N, sc_s[:, :], _NEG_INF)
        xmin = x1_s[:, :]
        ymin = y1_s[:, :]
        xmax = x2_s[:, :]
        ymax = y2_s[:, :]
        area = (jnp.maximum(xmax - xmin, 0.0)
                * jnp.maximum(ymax - ymin, 0.0))

        flat = (jax.lax.broadcasted_iota(jnp.int32, shape, 0) * 128
                + jax.lax.broadcasted_iota(jnp.int32, shape, 1))
        sub16 = jax.lax.broadcasted_iota(jnp.int32, (16, 128), 0)
        lane16 = jax.lax.broadcasted_iota(jnp.int32, (16, 128), 1)
        sub96 = jax.lax.broadcasted_iota(jnp.int32, (96, 128), 0)
        lane96 = jax.lax.broadcasted_iota(jnp.int32, (96, 128), 1)
        out_acc = jnp.zeros((16, 128), jnp.float32)

        for t in range(_NUM_PRED):
            m = jnp.max(scores)
            ok = m > _NEG_INF
            okf = jnp.where(ok, 1.0, 0.0).astype(jnp.float32)
            i = jnp.min(jnp.where(scores == m, flat, jnp.int32(2 ** 30)))
            sel = flat == i

            def ext(x):
                return jnp.sum(jnp.where(sel, x, 0.0))

            bx1 = ext(xmin)
            by1 = ext(ymin)
            bx2 = ext(xmax)
            by2 = ext(ymax)

            tile = t_s[i // 128]
            eqc = (tile == m) & (lane96 == i % 128) & (sub96 < _NCLS)
            bcls = jnp.min(jnp.where(eqc, sub96, 127)).astype(jnp.float32)

            row = (jnp.where(lane16 == 0, bcls, 0.0)
                   + jnp.where(lane16 == 1, m, 0.0)
                   + jnp.where(lane16 == 2, bx1, 0.0)
                   + jnp.where(lane16 == 3, by1, 0.0)
                   + jnp.where(lane16 == 4, bx2, 0.0)
                   + jnp.where(lane16 == 5, by2, 0.0))
            out_acc = out_acc + okf * jnp.where(sub16 == t, row, 0.0)

            ix1 = jnp.maximum(xmin, bx1)
            iy1 = jnp.maximum(ymin, by1)
            ix2 = jnp.minimum(xmax, bx2)
            iy2 = jnp.minimum(ymax, by2)
            inter = (jnp.maximum(ix2 - ix1, 0.0)
                     * jnp.maximum(iy2 - iy1, 0.0))
            barea = (jnp.maximum(bx2 - bx1, 0.0)
                     * jnp.maximum(by2 - by1, 0.0))
            iou = inter / jnp.maximum(area + barea - inter, 1e-8)
            supp = ((iou > _IOU_T) | sel) & ok
            scores = jnp.where(supp, _NEG_INF, scores)

        o_ref[0] = out_acc


def kernel(y_pred):
    b, n, d = y_pred.shape
    out = pl.pallas_call(
        _body,
        grid=(b, _NCHUNK),
        in_specs=[pl.BlockSpec((1, _CHUNK, d), lambda i, j: (i, j, 0))],
        out_specs=pl.BlockSpec((1, 16, 128), lambda i, j: (i, 0, 0)),
        out_shape=jax.ShapeDtypeStruct((b, 16, 128), jnp.float32),
        scratch_shapes=[
            pltpu.VMEM((_ROWS, 128), jnp.float32),
            pltpu.VMEM((_ROWS, 128), jnp.float32),
            pltpu.VMEM((_ROWS, 128), jnp.float32),
            pltpu.VMEM((_ROWS, 128), jnp.float32),
            pltpu.VMEM((_ROWS, 128), jnp.float32),
            pltpu.VMEM((_ROWS, 96, 128), jnp.float32),
        ],
        compiler_params=pltpu.CompilerParams(
            dimension_semantics=("arbitrary", "arbitrary")),
    )(y_pred)
    return out[:, :_NUM_PRED, :6]


# speedup vs baseline: 1.1474x; 1.0549x over previous
import jax
import jax.numpy as jnp
from jax.experimental import pallas as pl
from jax.experimental.pallas import tpu as pltpu

_IMG = 512.0
_CONF_T = 0.5
_IOU_T = 0.35
_NUM_PRED = 10
_NCLS = 81
_N = 20000
_CHUNK = 20480
_NCHUNK = 1
_ROWS = 160
_NEG_INF = float("-inf")


def _body(y_ref, o_ref, sc_s, x1_s, y1_s, x2_s, y2_s, t_s):
    j = pl.program_id(1)
    liota = jax.lax.broadcasted_iota(jnp.int32, (1, 128), 1)

    _LAST_TILES = 157 - (_NCHUNK - 1) * (_CHUNK // 128)

    def _tile(k):
        yk = y_ref[0, k * 128:(k + 1) * 128, :]
        t = jnp.swapaxes(yk, 0, 1)
        row = j * (_CHUNK // 128) + k
        t_s[pl.ds(row, 1), 0:93, :] = t.reshape(1, 93, 128)

        s0 = t[0:1, :]
        m_rest = jnp.max(t[1:_NCLS, :], axis=0, keepdims=True)
        conf = jnp.maximum(m_rest, s0)

        ocx = t[81:82, :]
        ocy = t[82:83, :]
        ow = t[83:84, :]
        oh = t[84:85, :]
        acx = t[85:86, :]
        acy = t[86:87, :]
        aw = t[87:88, :]
        ah = t[88:89, :]
        v0 = t[89:90, :]
        v1 = t[90:91, :]
        v2 = t[91:92, :]
        v3 = t[92:93, :]

        cx = ocx * v0 * aw + acx
        cy = ocy * v1 * ah + acy
        w = jnp.exp(ow * v2) * aw
        h = jnp.exp(oh * v3) * ah

        fl = j * _CHUNK + k * 128 + liota
        valid = (m_rest > s0) & (conf >= _CONF_T) & (fl < _N)
        scores = jnp.where(valid, conf, _NEG_INF)

        sc_s[pl.ds(row, 1), :] = scores
        x1_s[pl.ds(row, 1), :] = (cx - 0.5 * w) * _IMG
        y1_s[pl.ds(row, 1), :] = (cy - 0.5 * h) * _IMG
        x2_s[pl.ds(row, 1), :] = (cx + 0.5 * w) * _IMG
        y2_s[pl.ds(row, 1), :] = (cy + 0.5 * h) * _IMG

    for k in range(_CHUNK // 128):
        if k < _LAST_TILES:
            _tile(k)
        else:
            pl.when(j < _NCHUNK - 1)(lambda k=k: _tile(k))

    @pl.when(j == _NCHUNK - 1)
    def _():
        shape = (_ROWS, 128)
        flat0 = (jax.lax.broadcasted_iota(jnp.int32, shape, 0) * 128
                 + jax.lax.broadcasted_iota(jnp.int32, shape, 1))
        scores = jnp.where(flat0 < _N, sc_s[:, :], _NEG_INF)
        xmin = x1_s[:, :]
        ymin = y1_s[:, :]
        xmax = x2_s[:, :]
        ymax = y2_s[:, :]
        area = (jnp.maximum(xmax - xmin, 0.0)
                * jnp.maximum(ymax - ymin, 0.0))

        flat = (jax.lax.broadcasted_iota(jnp.int32, shape, 0) * 128
                + jax.lax.broadcasted_iota(jnp.int32, shape, 1))
        sub16 = jax.lax.broadcasted_iota(jnp.int32, (16, 128), 0)
        lane16 = jax.lax.broadcasted_iota(jnp.int32, (16, 128), 1)
        sub96 = jax.lax.broadcasted_iota(jnp.int32, (96, 128), 0)
        lane96 = jax.lax.broadcasted_iota(jnp.int32, (96, 128), 1)
        out_acc = jnp.zeros((16, 128), jnp.float32)

        for t in range(_NUM_PRED):
            m = jnp.max(scores)
            ok = m > _NEG_INF
            okf = jnp.where(ok, 1.0, 0.0).astype(jnp.float32)
            i = jnp.min(jnp.where(scores == m, flat, jnp.int32(2 ** 30)))
            sel = flat == i

            def ext(x):
                return jnp.sum(jnp.where(sel, x, 0.0))

            bx1 = ext(xmin)
            by1 = ext(ymin)
            bx2 = ext(xmax)
            by2 = ext(ymax)

            tile = t_s[i // 128]
            eqc = (tile == m) & (lane96 == i % 128) & (sub96 < _NCLS)
            bcls = jnp.min(jnp.where(eqc, sub96, 127)).astype(jnp.float32)

            row = (jnp.where(lane16 == 0, bcls, 0.0)
                   + jnp.where(lane16 == 1, m, 0.0)
                   + jnp.where(lane16 == 2, bx1, 0.0)
                   + jnp.where(lane16 == 3, by1, 0.0)
                   + jnp.where(lane16 == 4, bx2, 0.0)
                   + jnp.where(lane16 == 5, by2, 0.0))
            out_acc = out_acc + okf * jnp.where(sub16 == t, row, 0.0)

            ix1 = jnp.maximum(xmin, bx1)
            iy1 = jnp.maximum(ymin, by1)
            ix2 = jnp.minimum(xmax, bx2)
            iy2 = jnp.minimum(ymax, by2)
            inter = (jnp.maximum(ix2 - ix1, 0.0)
                     * jnp.maximum(iy2 - iy1, 0.0))
            barea = (jnp.maximum(bx2 - bx1, 0.0)
                     * jnp.maximum(by2 - by1, 0.0))
            iou = inter / jnp.maximum(area + barea - inter, 1e-8)
            supp = ((iou > _IOU_T) | sel) & ok
            scores = jnp.where(supp, _NEG_INF, scores)

        o_ref[0] = out_acc


def kernel(y_pred):
    b, n, d = y_pred.shape
    out = pl.pallas_call(
        _body,
        grid=(b, _NCHUNK),
        in_specs=[pl.BlockSpec((1, _CHUNK, d), lambda i, j: (i, j, 0))],
        out_specs=pl.BlockSpec((1, 16, 128), lambda i, j: (i, 0, 0)),
        out_shape=jax.ShapeDtypeStruct((b, 16, 128), jnp.float32),
        scratch_shapes=[
            pltpu.VMEM((_ROWS, 128), jnp.float32),
            pltpu.VMEM((_ROWS, 128), jnp.float32),
            pltpu.VMEM((_ROWS, 128), jnp.float32),
            pltpu.VMEM((_ROWS, 128), jnp.float32),
            pltpu.VMEM((_ROWS, 128), jnp.float32),
            pltpu.VMEM((_ROWS, 96, 128), jnp.float32),
        ],
        compiler_params=pltpu.CompilerParams(
            dimension_semantics=("arbitrary", "arbitrary")),
    )(y_pred)
    return out[:, :_NUM_PRED, :6]
